# baseline (speedup 1.0000x reference)
import jax
import jax.numpy as jnp
from jax import lax
from jax.experimental import pallas as pl
from jax.experimental.pallas import tpu as pltpu
from jax.experimental.pallas import tpu_sc as plsc

BATCH = 16384
N_FIELDS = 100
INPUT_DIM = 1000000
NW = 32
BW = BATCH // NW
LANES = 16
GROUPS = BW // LANES


def _wide_body(xt_hbm, tab_hbm, bias_hbm, emb_hbm, out_hbm,
               idx_v, vals_v, sums_v, bias_v, sem, isem):
    c = lax.axis_index("c")
    s = lax.axis_index("s")
    wid = s * 2 + c
    b0 = pl.multiple_of(wid * BW, 8)

    icps = [
        pltpu.async_copy(xt_hbm.at[f, pl.ds(b0, BW)],
                         idx_v.at[pl.ds(f * BW, BW)], isem)
        for f in range(N_FIELDS)
    ]
    pltpu.sync_copy(bias_hbm, bias_v)
    for cp in icps:
        cp.wait()

    tab_row = tab_hbm.at[0]
    cps = [
        pltpu.async_copy(tab_row.at[idx_v.at[pl.ds(f * BW, BW)]],
                         vals_v.at[pl.ds(f * BW, BW)], sem)
        for f in range(N_FIELDS)
    ]
    for cp in cps:
        cp.wait()

    ecps = [
        pltpu.async_copy(vals_v.at[pl.ds(f * BW, BW)],
                         emb_hbm.at[f, 0, pl.ds(b0, BW)], isem)
        for f in range(N_FIELDS)
    ]

    bias_vec = bias_v[...]

    def group_body(g, _):
        col0 = g * LANES
        accs = [vals_v[pl.ds(a * BW + col0, LANES)] for a in range(4)]
        for f in range(4, N_FIELDS):
            accs[f % 4] = accs[f % 4] + vals_v[pl.ds(f * BW + col0, LANES)]
        sums_v[pl.ds(col0, LANES)] = (
            (accs[0] + accs[1]) + (accs[2] + accs[3]) + bias_vec)
        return 0

    lax.fori_loop(0, GROUPS, group_body, 0)
    pltpu.sync_copy(sums_v, out_hbm.at[0].at[pl.ds(b0, BW)])
    for cp in ecps:
        cp.wait()


def kernel(X, weight, bias):
    Xt = jnp.transpose(X)
    bias16 = jnp.broadcast_to(bias.astype(jnp.float32), (LANES,))
    mesh = plsc.VectorSubcoreMesh(
        core_axis_name="c", subcore_axis_name="s",
        num_cores=2, num_subcores=16)
    emb_t, out = pl.kernel(
        _wide_body,
        out_type=(
            jax.ShapeDtypeStruct((N_FIELDS, 1, BATCH), jnp.float32),
            jax.ShapeDtypeStruct((1, BATCH), jnp.float32),
        ),
        mesh=mesh,
        compiler_params=pltpu.CompilerParams(
            needs_layout_passes=False, skip_device_barrier=True),
        scratch_types=[
            pltpu.VMEM((N_FIELDS * BW,), jnp.int32),
            pltpu.VMEM((N_FIELDS * BW,), jnp.float32),
            pltpu.VMEM((BW,), jnp.float32),
            pltpu.VMEM((LANES,), jnp.float32),
            pltpu.SemaphoreType.DMA,
            pltpu.SemaphoreType.DMA,
        ],
    )(Xt, weight.reshape(1, INPUT_DIM), bias16)
    emb = jnp.transpose(emb_t, (2, 0, 1))
    return (out.reshape(BATCH, 1), emb)


# speedup vs baseline: 1.0028x; 1.0028x over previous
import jax
import jax.numpy as jnp
from jax import lax
from jax.experimental import pallas as pl
from jax.experimental.pallas import tpu as pltpu
from jax.experimental.pallas import tpu_sc as plsc

BATCH = 16384
N_FIELDS = 100
INPUT_DIM = 1000000
NW = 32
BW = BATCH // NW
LANES = 16
GROUPS = BW // LANES


def _wide_body(xt_hbm, tab_hbm, bias_hbm, emb_hbm, out_hbm,
               idx_v, vals_v, sums_v, bias_v, sem, isem):
    c = lax.axis_index("c")
    s = lax.axis_index("s")
    wid = s * 2 + c
    b0 = pl.multiple_of(wid * BW, 8)

    icps = [
        pltpu.async_copy(xt_hbm.at[f, pl.ds(b0, BW)],
                         idx_v.at[pl.ds(f * BW, BW)], isem)
        for f in range(N_FIELDS)
    ]
    pltpu.sync_copy(bias_hbm, bias_v)
    for cp in icps:
        cp.wait()

    tab_row = tab_hbm.at[0]
    cps = [
        pltpu.async_copy(tab_row.at[idx_v.at[pl.ds(f * BW, BW)]],
                         vals_v.at[pl.ds(f * BW, BW)], sem)
        for f in range(N_FIELDS)
    ]
    for cp in cps:
        cp.wait()

    ecps = [
        pltpu.async_copy(vals_v.at[pl.ds(f * BW, BW)],
                         emb_hbm.at[f, 0, pl.ds(b0, BW)], isem)
        for f in range(N_FIELDS)
    ]

    bias_vec = bias_v[...]

    def group_body(g, _):
        col0 = g * LANES
        accs = [vals_v[pl.ds(a * BW + col0, LANES)] for a in range(4)]
        for f in range(4, N_FIELDS):
            accs[f % 4] = accs[f % 4] + vals_v[pl.ds(f * BW + col0, LANES)]
        sums_v[pl.ds(col0, LANES)] = (
            (accs[0] + accs[1]) + (accs[2] + accs[3]) + bias_vec)
        return 0

    lax.fori_loop(0, GROUPS, group_body, 0)
    pltpu.sync_copy(sums_v, out_hbm.at[0].at[pl.ds(b0, BW)])
    for cp in ecps:
        cp.wait()


def kernel(X, weight, bias):
    Xt = jnp.transpose(X)
    bias16 = jnp.broadcast_to(bias.astype(jnp.float32), (LANES,))
    mesh = plsc.VectorSubcoreMesh(
        core_axis_name="c", subcore_axis_name="s",
        num_cores=2, num_subcores=16)
    emb_t, out = pl.kernel(
        _wide_body,
        out_type=(
            jax.ShapeDtypeStruct((N_FIELDS, 1, BATCH), jnp.float32),
            jax.ShapeDtypeStruct((1, BATCH), jnp.float32),
        ),
        mesh=mesh,
        compiler_params=pltpu.CompilerParams(needs_layout_passes=False),
        scratch_types=[
            pltpu.VMEM((N_FIELDS * BW,), jnp.int32),
            pltpu.VMEM((N_FIELDS * BW,), jnp.float32),
            pltpu.VMEM((BW,), jnp.float32),
            pltpu.VMEM((LANES,), jnp.float32),
            pltpu.SemaphoreType.DMA,
            pltpu.SemaphoreType.DMA,
        ],
    )(Xt, weight.reshape(1, INPUT_DIM), bias16)
    emb = jnp.transpose(emb_t, (2, 0, 1))
    return (out.reshape(BATCH, 1), emb)
